# baseline (speedup 1.0000x reference)
import functools
import jax
import jax.numpy as jnp
from jax import lax
from jax.experimental import pallas as pl
from jax.experimental.pallas import tpu as pltpu
from jax.experimental.pallas import tpu_sc as plsc

B, L, V, D, H, C = 4096, 50, 100000, 128, 128, 20

NC, NS = 2, 16
NW = NC * NS
BW = B // NW
NB = 4
NCHUNK = BW // NB
GOFF = (0, 104)
GLEN = (104, 96)


def _emb_sum_body(table_hbm, idx_hbm, par_hbm, out_hbm, idx_v, par_v,
                  rows0, rows1, out_all, spar, mpar, sem0, sem1):
    sid = lax.axis_index("s")
    wid = sid * NC + lax.axis_index("c")
    flat_base = wid * (BW * L)
    row_base = wid * BW

    pltpu.sync_copy(idx_hbm.at[pl.ds(flat_base, BW * L)], idx_v)
    pltpu.sync_copy(par_hbm.at[pl.ds(flat_base, BW * L)], par_v)

    rows = (rows0, rows1)
    sems = (sem0, sem1)

    def fire(i, p):
        for off, n in zip(GOFF, GLEN):
            pltpu.async_copy(
                table_hbm.at[idx_v.at[pl.ds(i * (NB * L) + off, n)]],
                rows[p].at[pl.ds(off, n)],
                sems[p],
            )

    def drain(p):
        pltpu.make_async_copy(
            table_hbm.at[pl.ds(0, NB * L)], rows[p], sems[p]
        ).wait()

    def reduce(i, p):
        rbuf = rows[p]
        myspar = spar.at[pl.ds(sid * (NB * L), NB * L)]
        pltpu.sync_copy(par_v.at[pl.ds(i * (NB * L), NB * L)], myspar)
        pltpu.sync_copy(myspar, mpar)
        G = 10
        for b in range(NB):
            base = b * L
            row = i * NB + b
            for g in range(L // G):
                accs = []
                j0 = base + g * G
                q = mpar[j0]
                for k in range(D // 32):
                    x = plsc.bitcast(rbuf[j0, pl.ds(q + k * 16, 16)],
                                     jnp.bfloat16)
                    accs.append(
                        plsc.unpack(x, format=plsc.PackFormat.INTERLEAVED))
                for l in range(1, G):
                    q = mpar[j0 + l]
                    for k in range(D // 32):
                        x = plsc.bitcast(
                            rbuf[j0 + l, pl.ds(q + k * 16, 16)],
                            jnp.bfloat16)
                        pa, pb = plsc.unpack(
                            x, format=plsc.PackFormat.INTERLEAVED)
                        accs[k] = (accs[k][0] + pa, accs[k][1] + pb)
                for k in range(D // 32):
                    sla = pl.ds(k * 32, 16)
                    slb = pl.ds(k * 32 + 16, 16)
                    if g == 0:
                        out_all[row, sla] = accs[k][0]
                        out_all[row, slb] = accs[k][1]
                    else:
                        plsc.addupdate(out_all.at[row, sla], accs[k][0])
                        plsc.addupdate(out_all.at[row, slb], accs[k][1])

    fire(0, 0)

    def pair(j, carry):
        i0 = 2 * j
        fire(i0 + 1, 1)
        drain(0)
        reduce(i0, 0)

        @pl.when(j < (NCHUNK // 2) - 1)
        def _():
            fire(i0 + 2, 0)

        drain(1)
        reduce(i0 + 1, 1)
        return carry

    lax.fori_loop(0, NCHUNK // 2, pair, 0)
    pltpu.sync_copy(out_all, out_hbm.at[pl.ds(row_base, BW)])


@functools.partial(
    pl.kernel,
    mesh=plsc.VectorSubcoreMesh(core_axis_name="c", subcore_axis_name="s"),
    out_type=jax.ShapeDtypeStruct((B, D), jnp.float32),
    compiler_params=pltpu.CompilerParams(needs_layout_passes=False),
    scratch_types=[
        pltpu.VMEM((BW * L,), jnp.int32),
        pltpu.VMEM((BW * L,), jnp.int32),
        pltpu.VMEM((NB * L, D), jnp.int32),
        pltpu.VMEM((NB * L, D), jnp.int32),
        pltpu.VMEM((BW, D), jnp.float32),
        pltpu.VMEM_SHARED((NS * NB * L,), jnp.int32),
        pltpu.SMEM((NB * L,), jnp.int32),
        pltpu.SemaphoreType.DMA,
        pltpu.SemaphoreType.DMA,
    ],
)
def _emb_sum(table_hbm, idx_hbm, par_hbm, out_hbm, idx_v, par_v,
             rows0, rows1, out_all, spar, mpar, sem0, sem1):
    _emb_sum_body(table_hbm, idx_hbm, par_hbm, out_hbm, idx_v, par_v,
                  rows0, rows1, out_all, spar, mpar, sem0, sem1)


def _mlp_body(sums_ref, idx_ref, w1_ref, b1_ref, w2_ref, b2_ref, out_ref):
    s = sums_ref[...]
    idxb = idx_ref[...]
    cnt = jnp.sum((idxb != 0).astype(jnp.float32), axis=1, keepdims=True)
    mean = s * (1.0 / jnp.maximum(cnt, 1.0))
    h = lax.dot_general(
        mean, w1_ref[...], (((1,), (1,)), ((), ())),
        preferred_element_type=jnp.float32,
    ) + b1_ref[...]
    h = jnp.maximum(h, 0.0)
    out = lax.dot_general(
        h, w2_ref[...], (((1,), (1,)), ((), ())),
        preferred_element_type=jnp.float32,
    ) + b2_ref[...]
    out_ref[...] = out


def kernel(input_features_b_l, input_length_b, table, W1, b1, W2, b2):
    del input_length_b
    idx = input_features_b_l.astype(jnp.int32)
    idx_flat = idx.reshape(-1)
    gidx_flat = idx_flat >> 1
    par_flat = (idx_flat & 1) * 64
    tbl16 = table.astype(jnp.bfloat16)
    tbl_pack = lax.bitcast_convert_type(
        tbl16.reshape(V // 2, D, 2), jnp.int32)
    sums = _emb_sum(tbl_pack, gidx_flat, par_flat)
    w1r = W1.reshape(H, D // 32, 16, 2)
    w1p = jnp.concatenate([w1r[:, :, :, 0], w1r[:, :, :, 1]],
                          axis=-1).reshape(H, D)
    out = pl.pallas_call(
        _mlp_body,
        out_shape=jax.ShapeDtypeStruct((B, C), jnp.float32),
    )(sums, idx, w1p, b1.reshape(1, H), W2, b2.reshape(1, C))
    return out


# speedup vs baseline: 23.4229x; 23.4229x over previous
import functools
import jax
import jax.numpy as jnp
from jax import lax
from jax.experimental import pallas as pl
from jax.experimental.pallas import tpu as pltpu
from jax.experimental.pallas import tpu_sc as plsc

B, L, V, D, H, C = 4096, 50, 100000, 128, 128, 20

NC, NS = 2, 16
NW = NC * NS
BW = B // NW
NB = 4
NCHUNK = BW // NB
GOFF = (0, 104)
GLEN = (104, 96)


def _emb_sum_body(table_hbm, idx_hbm, par_hbm, out_hbm, idx_v, par_v,
                  rows0, rows1, out_all, spar, mpar, sem0, sem1):
    sid = lax.axis_index("s")
    wid = sid * NC + lax.axis_index("c")
    flat_base = wid * (BW * L)
    row_base = wid * BW

    pltpu.sync_copy(idx_hbm.at[pl.ds(flat_base, BW * L)], idx_v)
    pltpu.sync_copy(par_hbm.at[pl.ds(flat_base, BW * L)], par_v)

    rows = (rows0, rows1)
    sems = (sem0, sem1)

    def fire(i, p):
        for off, n in zip(GOFF, GLEN):
            pltpu.async_copy(
                table_hbm.at[idx_v.at[pl.ds(i * (NB * L) + off, n)]],
                rows[p].at[pl.ds(off, n)],
                sems[p],
            )

    def drain(p):
        pltpu.make_async_copy(
            table_hbm.at[pl.ds(0, NB * L)], rows[p], sems[p]
        ).wait()

    def reduce(i, p):
        rbuf = rows[p]
        myspar = spar.at[pl.ds(sid * (NB * L), NB * L)]
        pltpu.sync_copy(par_v.at[pl.ds(i * (NB * L), NB * L)], myspar)
        pltpu.sync_copy(myspar, mpar)
        G = 10
        for b in range(NB):
            base = b * L
            row = i * NB + b
            for g in range(L // G):
                accs = []
                j0 = base + g * G
                q = mpar[j0]
                for k in range(D // 32):
                    x = plsc.bitcast(rbuf[j0, pl.ds(q + k * 16, 16)],
                                     jnp.bfloat16)
                    accs.append(
                        plsc.unpack(x, format=plsc.PackFormat.INTERLEAVED))
                for l in range(1, G):
                    q = mpar[j0 + l]
                    for k in range(D // 32):
                        x = plsc.bitcast(
                            rbuf[j0 + l, pl.ds(q + k * 16, 16)],
                            jnp.bfloat16)
                        pa, pb = plsc.unpack(
                            x, format=plsc.PackFormat.INTERLEAVED)
                        accs[k] = (accs[k][0] + pa, accs[k][1] + pb)
                for k in range(D // 32):
                    sla = pl.ds(k * 16, 16)
                    slb = pl.ds(D // 2 + k * 16, 16)
                    if g == 0:
                        out_all[row, sla] = accs[k][0]
                        out_all[row, slb] = accs[k][1]
                    else:
                        plsc.addupdate(out_all.at[row, sla], accs[k][0])
                        plsc.addupdate(out_all.at[row, slb], accs[k][1])

    fire(0, 0)

    def pair(j, carry):
        i0 = 2 * j
        fire(i0 + 1, 1)
        drain(0)
        reduce(i0, 0)

        @pl.when(j < (NCHUNK // 2) - 1)
        def _():
            fire(i0 + 2, 0)

        drain(1)
        reduce(i0 + 1, 1)
        return carry

    lax.fori_loop(0, NCHUNK // 2, pair, 0)
    pltpu.sync_copy(out_all, out_hbm.at[pl.ds(row_base, BW)])


@functools.partial(
    pl.kernel,
    mesh=plsc.VectorSubcoreMesh(core_axis_name="c", subcore_axis_name="s"),
    out_type=jax.ShapeDtypeStruct((B, D), jnp.float32),
    compiler_params=pltpu.CompilerParams(needs_layout_passes=False),
    scratch_types=[
        pltpu.VMEM((BW * L,), jnp.int32),
        pltpu.VMEM((BW * L,), jnp.int32),
        pltpu.VMEM((NB * L, D), jnp.int32),
        pltpu.VMEM((NB * L, D), jnp.int32),
        pltpu.VMEM((BW, D), jnp.float32),
        pltpu.VMEM_SHARED((NS * NB * L,), jnp.int32),
        pltpu.SMEM((NB * L,), jnp.int32),
        pltpu.SemaphoreType.DMA,
        pltpu.SemaphoreType.DMA,
    ],
)
def _emb_sum(table_hbm, idx_hbm, par_hbm, out_hbm, idx_v, par_v,
             rows0, rows1, out_all, spar, mpar, sem0, sem1):
    _emb_sum_body(table_hbm, idx_hbm, par_hbm, out_hbm, idx_v, par_v,
                  rows0, rows1, out_all, spar, mpar, sem0, sem1)


def _mlp_body(sums_ref, idx_ref, w1_ref, b1_ref, w2_ref, b2_ref, out_ref):
    s = sums_ref[...]
    idxb = idx_ref[...]
    cnt = jnp.sum((idxb != 0).astype(jnp.float32), axis=1, keepdims=True)
    mean = s * (1.0 / jnp.maximum(cnt, 1.0))
    h = lax.dot_general(
        mean, w1_ref[...], (((1,), (1,)), ((), ())),
        preferred_element_type=jnp.float32,
    ) + b1_ref[...]
    h = jnp.maximum(h, 0.0)
    out = lax.dot_general(
        h, w2_ref[...], (((1,), (1,)), ((), ())),
        preferred_element_type=jnp.float32,
    ) + b2_ref[...]
    out_ref[...] = out


def kernel(input_features_b_l, input_length_b, table, W1, b1, W2, b2):
    del input_length_b
    idx = input_features_b_l.astype(jnp.int32)
    idx_flat = idx.reshape(-1)
    gidx_flat = idx_flat >> 1
    par_flat = (idx_flat & 1) * 64
    t16 = lax.bitcast_convert_type(table.astype(jnp.bfloat16), jnp.uint16)
    lo = t16[:, :D // 2].astype(jnp.int32)
    hi = t16[:, D // 2:].astype(jnp.int32)
    tbl_pack = (lo | (hi << 16)).reshape(V // 2, D)
    sums = _emb_sum(tbl_pack, gidx_flat, par_flat)
    out = pl.pallas_call(
        _mlp_body,
        out_shape=jax.ShapeDtypeStruct((B, C), jnp.float32),
    )(sums, idx, W1, b1.reshape(1, H), W2, b2.reshape(1, C))
    return out
